# baseline (speedup 1.0000x reference)
import functools

import jax
import jax.numpy as jnp
from jax import lax
from jax.experimental import pallas as pl
from jax.experimental.pallas import tpu as pltpu
from jax.experimental.pallas import tpu_sc as plsc

EMBED_DIM = 1024
NC = 2
NS = 16
NW = NC * NS
CHUNK = 32
NBUF = 3


def _emb_body(b_per_w, n_chunks, w_per_row, table_hbm, tok_hbm, out_hbm,
              idx_v, rows_v, *sems):
    g_sems, o_sems = sems[:NBUF], sems[NBUF:]
    wid = lax.axis_index("s") * NC + lax.axis_index("c")
    base = wid * b_per_w
    row = wid // w_per_row
    col = (wid % w_per_row) * b_per_w
    pltpu.sync_copy(tok_hbm.at[row, pl.ds(col, b_per_w)], idx_v)

    def gather(i, b):
        return pltpu.async_copy(
            table_hbm.at[idx_v.at[pl.ds(i * CHUNK, CHUNK)]],
            rows_v.at[b], g_sems[b])

    def writeback(i, b):
        return pltpu.async_copy(
            rows_v.at[b], out_hbm.at[pl.ds(base + i * CHUNK, CHUNK)],
            o_sems[b])

    g_cp = [None] * NBUF
    o_cp = [None] * NBUF
    for b in range(min(NBUF, n_chunks)):
        g_cp[b] = gather(b, b)
    for i in range(n_chunks):
        b = i % NBUF
        g_cp[b].wait()
        o_cp[b] = writeback(i, b)
        nxt = i + NBUF
        if nxt < n_chunks:
            o_cp[b].wait()
            g_cp[b] = gather(nxt, b)
    for i in range(max(0, n_chunks - NBUF), n_chunks):
        o_cp[i % NBUF].wait()


@functools.partial(jax.jit, static_argnames=("batch", "seq"))
def _embed(table, tokens, batch, seq):
    n_tok = batch * seq
    b_per_w = n_tok // NW
    n_chunks = b_per_w // CHUNK
    w_per_row = seq // b_per_w
    mesh = plsc.VectorSubcoreMesh(core_axis_name="c", subcore_axis_name="s")
    kern = pl.kernel(
        functools.partial(_emb_body, b_per_w, n_chunks, w_per_row),
        mesh=mesh,
        out_type=jax.ShapeDtypeStruct((n_tok, EMBED_DIM), jnp.float32),
        scratch_types=[
            pltpu.VMEM((b_per_w,), jnp.int32),
            pltpu.VMEM((NBUF, CHUNK, EMBED_DIM), jnp.float32),
        ] + [pltpu.SemaphoreType.DMA] * (2 * NBUF),
    )
    return kern(table, tokens)


def kernel(tokens, embed_table):
    batch, seq = tokens.shape
    out = _embed(embed_table, tokens, batch, seq)
    return out.reshape(batch, seq, EMBED_DIM)
